# Initial kernel scaffold; baseline (speedup 1.0000x reference)
import jax
import jax.numpy as jnp
from jax.experimental import pallas as pl


def kernel(h, edge_index, Wc0, bc0, Wc1, bc1, Wc2, bc2, Wg0, bg0, Wg1, bg1, Wg2, bg2, Wg3, bg3, Wp0, bp0, Wp1, bp1, Wp2, bp2, Wcls, bcls):
    raise NotImplementedError("write your pallas kernel here")


# speedup vs baseline: 4.0297x; 4.0297x over previous
import functools

import jax
import jax.numpy as jnp
from jax import lax
from jax.experimental import pallas as pl
from jax.experimental.pallas import tpu as pltpu
from jax.experimental.pallas import tpu_sc as plsc

N = 10000
DH = 128
DOUT = 64
E = 320000

L = 128
NC = 2
NS = 16
NW = NC * NS
RPW = 79
EPAD = NW * RPW * L
NP = RPW * L
ZR = NP // NS
SCRAP = N

_F32 = jnp.float32


def _sc_mesh():
    return plsc.VectorSubcoreMesh(
        core_axis_name="c", subcore_axis_name="s", num_cores=NC, num_subcores=NS
    )


@functools.partial(
    pl.kernel,
    out_type=(
        jax.ShapeDtypeStruct((NC, RPW, L), _F32),
        jax.ShapeDtypeStruct((NC, RPW, L), _F32),
    ),
    mesh=_sc_mesh(),
    scratch_types=[
        pltpu.VMEM((RPW, L), jnp.int32),
        pltpu.VMEM((RPW, L), jnp.int32),
        pltpu.VMEM((RPW, L), _F32),
        pltpu.VMEM((RPW, L), _F32),
        pltpu.VMEM((RPW,), jnp.int32),
        pltpu.VMEM_SHARED((RPW, L), _F32),
        pltpu.VMEM_SHARED((RPW, L), _F32),
    ],
    compiler_params=pltpu.CompilerParams(needs_layout_passes=False),
)
def _deg_kernel(src_hbm, dst_hbm, zsheet_hbm, rowids_hbm, dsrc_out, ddst_out,
                src_v, dst_v, asrc_v, adst_v, rowids_v, sh_src, sh_dst):
    cid = lax.axis_index("c")
    sid = lax.axis_index("s")
    wid = sid * NC + cid
    pltpu.sync_copy(zsheet_hbm, asrc_v)
    pltpu.sync_copy(zsheet_hbm, adst_v)
    pltpu.sync_copy(src_hbm.at[wid], src_v)
    pltpu.sync_copy(dst_hbm.at[wid], dst_v)
    pltpu.sync_copy(rowids_hbm, rowids_v)

    @pl.when(sid == 0)
    def _():
        pltpu.sync_copy(zsheet_hbm, sh_src)
        pltpu.sync_copy(zsheet_hbm, sh_dst)

    ones = jnp.ones((16,), _F32)

    def step(j, carry):
        for k in range(8):
            s16 = src_v[j, pl.ds(16 * k, 16)]
            plsc.addupdate_scatter(
                asrc_v,
                [lax.shift_right_logical(s16, 7), lax.bitwise_and(s16, 127)],
                ones)
            d16 = dst_v[j, pl.ds(16 * k, 16)]
            plsc.addupdate_scatter(
                adst_v,
                [lax.shift_right_logical(d16, 7), lax.bitwise_and(d16, 127)],
                ones)
        return carry

    lax.fori_loop(0, RPW, step, 0)
    plsc.subcore_barrier()
    pltpu.sync_copy(asrc_v, sh_src.at[rowids_v], add=True)
    pltpu.sync_copy(adst_v, sh_dst.at[rowids_v], add=True)
    plsc.subcore_barrier()

    @pl.when(sid == 0)
    def _():
        pltpu.sync_copy(sh_src, dsrc_out.at[cid])
        pltpu.sync_copy(sh_dst, ddst_out.at[cid])


@functools.partial(
    pl.kernel,
    out_type=jax.ShapeDtypeStruct((NC, NP, DH), _F32),
    mesh=_sc_mesh(),
    scratch_types=[
        pltpu.VMEM((RPW, L), jnp.int32),
        pltpu.VMEM((RPW, L), jnp.int32),
        pltpu.VMEM((L, DH), _F32),
        pltpu.VMEM_SHARED((NP, DH), _F32),
        pltpu.SemaphoreType.DMA,
    ],
)
def _agg_kernel(xs_hbm, src_hbm, dst_hbm, zrow_hbm, parts_out,
                src_v, dst_v, rows_v, agg_sh, sem):
    cid = lax.axis_index("c")
    sid = lax.axis_index("s")
    wid = sid * NC + cid
    pltpu.sync_copy(zrow_hbm, agg_sh.at[pl.ds(sid * ZR, ZR)])
    pltpu.sync_copy(src_hbm.at[wid], src_v)
    pltpu.sync_copy(dst_hbm.at[wid], dst_v)
    plsc.subcore_barrier()

    def step(j, carry):
        pltpu.async_copy(xs_hbm.at[src_v.at[j]], rows_v, sem).wait()
        pltpu.sync_copy(rows_v, agg_sh.at[dst_v.at[j]], add=True)
        return carry

    lax.fori_loop(0, RPW, step, 0)
    plsc.subcore_barrier()
    pltpu.sync_copy(agg_sh.at[pl.ds(sid * ZR, ZR)],
                    parts_out.at[cid, pl.ds(sid * ZR, ZR)])


def _att_pool(x, wg, wp, bp):
    g = jnp.dot(x, wg, preferred_element_type=_F32)
    m = jnp.max(g, axis=0, keepdims=True)
    e = jnp.exp(g - m)
    s = jnp.sum(e, axis=0, keepdims=True)
    pooled = jnp.sum((e / s) * x, axis=0, keepdims=True)
    return jnp.dot(pooled, wp, preferred_element_type=_F32) + bp


def _prologue_body(h_ref, dsrc_ref, ddst_ref, wg_ref, wp_ref, bp_ref,
                   xs_ref, rso_ref, rsi_ref, h0_ref):
    dsrc = dsrc_ref[0] + dsrc_ref[1]
    ddst = ddst_ref[0] + ddst_ref[1]
    rso = lax.rsqrt(jnp.maximum(dsrc, 1.0))
    rsi = lax.rsqrt(jnp.maximum(ddst, 1.0))
    rso_ref[...] = rso
    rsi_ref[...] = rsi
    x = h_ref[...]
    xs_ref[pl.ds(0, N), :] = x * rso[:N]
    xs_ref[pl.ds(N, NP - N), :] = jnp.zeros((NP - N, DH), _F32)
    h0_ref[...] = _att_pool(x, wg_ref[...], wp_ref[...], bp_ref[...])


_prologue_call = pl.pallas_call(
    _prologue_body,
    out_shape=(
        jax.ShapeDtypeStruct((NP, DH), _F32),
        jax.ShapeDtypeStruct((NP, 1), _F32),
        jax.ShapeDtypeStruct((NP, 1), _F32),
        jax.ShapeDtypeStruct((1, DOUT), _F32),
    ),
)


def _layer_body(p_ref, rsi_ref, rso_ref, wc_ref, bc_ref, wg_ref, wp_ref,
                bp_ref, xs_ref, h_ref):
    agg = (p_ref[0] + p_ref[1]) * rsi_ref[...]
    x = jnp.maximum(
        jnp.dot(agg, wc_ref[...], preferred_element_type=_F32) + bc_ref[...],
        0.0)
    xs_ref[...] = x * rso_ref[...]
    h_ref[...] = _att_pool(x[:N], wg_ref[...], wp_ref[...], bp_ref[...])


_layer_call = pl.pallas_call(
    _layer_body,
    out_shape=(
        jax.ShapeDtypeStruct((NP, DH), _F32),
        jax.ShapeDtypeStruct((1, DOUT), _F32),
    ),
)


def _final_body(p_ref, rsi_ref, wc_ref, bc_ref, wg_ref, wp_ref, bp_ref,
                h0_ref, h1_ref, h2_ref, out_ref):
    agg = (p_ref[0] + p_ref[1]) * rsi_ref[...]
    x = jnp.maximum(
        jnp.dot(agg, wc_ref[...], preferred_element_type=_F32) + bc_ref[...],
        0.0)
    h3 = _att_pool(x[:N], wg_ref[...], wp_ref[...], bp_ref[...])
    out_ref[...] = (h0_ref[...] + h1_ref[...] + h2_ref[...] + h3) * 0.25


_final_call = pl.pallas_call(
    _final_body,
    out_shape=jax.ShapeDtypeStruct((1, DOUT), _F32),
)


def kernel(h, edge_index, Wc0, bc0, Wc1, bc1, Wc2, bc2, Wg0, bg0, Wg1, bg1,
           Wg2, bg2, Wg3, bg3, Wp0, bp0, Wp1, bp1, Wp2, bp2, Wcls, bcls):
    del bg0, bg1, bg2, bg3
    pad = jnp.full((EPAD - E,), SCRAP, jnp.int32)
    src_r = jnp.concatenate([edge_index[0], pad]).reshape(NW, RPW, L)
    dst_r = jnp.concatenate([edge_index[1], pad]).reshape(NW, RPW, L)
    zsheet = jnp.zeros((RPW, L), _F32)
    zrow = jnp.zeros((ZR, DH), _F32)

    rowids = jnp.arange(RPW, dtype=jnp.int32)
    dsrc_p, ddst_p = _deg_kernel(src_r, dst_r, zsheet, rowids)
    xs, rso, rsi, h0 = _prologue_call(
        h, dsrc_p.reshape(NC, NP, 1), ddst_p.reshape(NC, NP, 1),
        Wg0, Wp0, bp0.reshape(1, DOUT))

    parts = _agg_kernel(xs, src_r, dst_r, zrow)
    xs, h1 = _layer_call(parts, rsi, rso, Wc0, bc0.reshape(1, DH), Wg1,
                         Wp1, bp1.reshape(1, DOUT))
    parts = _agg_kernel(xs, src_r, dst_r, zrow)
    xs, h2 = _layer_call(parts, rsi, rso, Wc1, bc1.reshape(1, DH), Wg2,
                         Wp2, bp2.reshape(1, DOUT))
    parts = _agg_kernel(xs, src_r, dst_r, zrow)
    return _final_call(parts, rsi, Wc2, bc2.reshape(1, DH), Wg3, Wcls,
                       bcls.reshape(1, DOUT), h0, h1, h2)
